# baseline (speedup 1.0000x reference)
import numpy as np
import jax
import jax.numpy as jnp
from jax.experimental import pallas as pl

_C, _T, _H, _W = 3, 32, 256, 256
_ALPHA = 4
_NSLOW = _T // _ALPHA
_IDX = np.linspace(0.0, _T - 1, _NSLOW).astype(np.int32)
_HB = 64
_NQ = _H // _HB


def _body(in_ref, slow_ref, fast_ref):
    fast_ref[...] = in_ref[...]
    for s in range(_NSLOW):
        slow_ref[:, pl.ds(s, 1)] = in_ref[:, pl.ds(int(_IDX[s]), 1)]


def kernel(frames):
    slow, fast = pl.pallas_call(
        _body,
        grid=(_NQ,),
        in_specs=[pl.BlockSpec((_C, _T, _HB, _W), lambda q: (0, 0, q, 0))],
        out_specs=[
            pl.BlockSpec((_C, _NSLOW, _HB, _W), lambda q: (0, 0, q, 0)),
            pl.BlockSpec((_C, _T, _HB, _W), lambda q: (0, 0, q, 0)),
        ],
        out_shape=[
            jax.ShapeDtypeStruct((_C, _NSLOW, _H, _W), jnp.float32),
            jax.ShapeDtypeStruct((_C, _T, _H, _W), jnp.float32),
        ],
    )(frames)
    return (slow, fast)
